# Initial kernel scaffold; baseline (speedup 1.0000x reference)
import jax
import jax.numpy as jnp
from jax.experimental import pallas as pl


def kernel(x0, x1, center0, center1, params, size0, size1):
    raise NotImplementedError("write your pallas kernel here")


# speedup vs baseline: 1.7988x; 1.7988x over previous
import jax
import jax.numpy as jnp
import numpy as np
from jax.experimental import pallas as pl
from jax.experimental.pallas import tpu as pltpu

_CPARAMS = pltpu.CompilerParams(vmem_limit_bytes=67_000_000)

C = 256
HD = 256
FC = 8
SC = HD // FC
H = 64
W = 64
L = H * W
HC = H // 4
WC = W // 4
S = HC * WC
WP = W + 2
HP = H + 2
LP = HP * WP

_PREC = jax.lax.Precision.HIGHEST
_NCH = 8
_CHUNK = L // _NCH


def _linspace32(b, n):
    v = np.arange(n, dtype=np.float32) * (np.float32(b)
                                          * (np.float32(1.0) / np.float32(n - 1)))
    v[-1] = np.float32(b)
    return v


def _interp_tables(hi, ho):
    ys = _linspace32(float(hi - 1), ho)
    y0 = np.floor(ys).astype(np.int64)
    y1 = np.minimum(y0 + 1, hi - 1).astype(np.int64)
    wy = (ys - y0.astype(np.float32)).astype(np.float32)
    wyc = (np.float32(1.0) - wy).astype(np.float32)
    return y0, y1, wy, wyc


_UY0, _UY1, _UWY, _UWYC = _interp_tables(HC, H)
_DY0, _DY1, _DWY, _DWYC = _interp_tables(H, HC)


def _bdot(a, b, dims=None):
    if dims is None:
        dims = (((a.ndim - 1,), (0,)), ((), ()))
    return jax.lax.dot_general(a, b, dims,
                               preferred_element_type=jnp.float32)


def _full(a):
    return pl.BlockSpec(a.shape, lambda i: (0,) * a.ndim)


def _merge_a_body(x_ref, c_ref, wx_ref, wxc_ref, w0_ref, b0_ref, t_ref):
    cent = c_ref[0]
    cimg = cent.reshape(HC, WC, C)
    cg0 = jnp.concatenate(
        [cimg[:, _UY0[w]:_UY0[w] + 1, :] for w in range(W)], axis=1)
    cg1 = jnp.concatenate(
        [cimg[:, _UY1[w]:_UY1[w] + 1, :] for w in range(W)], axis=1)
    wx = wx_ref[...]
    wxc = wxc_ref[...]
    rows = _CHUNK // W
    for c in range(_NCH):
        ups = []
        for j in range(rows):
            hh = c * rows + j
            a = cg0[_UY0[hh]]
            b_ = cg1[_UY0[hh]]
            c_ = cg0[_UY1[hh]]
            d_ = cg1[_UY1[hh]]
            top = a * wxc + b_ * wx
            bot = c_ * wxc + d_ * wx
            ups.append(top * _UWYC[hh] + bot * _UWY[hh])
        up = jnp.concatenate(ups, axis=0)
        r = slice(c * _CHUNK, (c + 1) * _CHUNK)
        t_ref[0, r, :] = _bdot(
            jnp.concatenate([x_ref[0, r, :], up], axis=1),
            w0_ref[...]) + b0_ref[...]


def _merge_a(xs, cs, mp):
    b = xs.shape[0]
    w0 = mp['lin0']['W']
    wxu = jnp.asarray(_UWY.reshape(W, 1))
    wxcu = jnp.asarray(_UWYC.reshape(W, 1))
    return pl.pallas_call(
        _merge_a_body,
        grid=(b,),
        compiler_params=_CPARAMS,
        in_specs=[
            pl.BlockSpec((1, L, C), lambda i: (i, 0, 0)),
            pl.BlockSpec((1, S, C), lambda i: (i, 0, 0)),
            _full(wxu), _full(wxcu), _full(w0), _full(mp['lin0']['b']),
        ],
        out_specs=pl.BlockSpec((1, L, 2 * C), lambda i: (i, 0, 0)),
        out_shape=jax.ShapeDtypeStruct((b, L, 2 * C), jnp.float32),
    )(xs, cs, wxu, wxcu, w0, mp['lin0']['b'])


def _merge_b1_body(tg_ref, w1_ref, b1_ref, h_ref):
    for c in range(_NCH):
        r = slice(c * _CHUNK, (c + 1) * _CHUNK)
        h_ref[0, r, :] = _bdot(tg_ref[0, r, :], w1_ref[...]) + b1_ref[...]


def _merge_b1(tg, mp):
    b = tg.shape[0]
    return pl.pallas_call(
        _merge_b1_body,
        grid=(b,),
        compiler_params=_CPARAMS,
        in_specs=[
            pl.BlockSpec((1, L, 2 * C), lambda i: (i, 0, 0)),
            _full(mp['lin1']['W']), _full(mp['lin1']['b']),
        ],
        out_specs=pl.BlockSpec((1, L, C), lambda i: (i, 0, 0)),
        out_shape=jax.ShapeDtypeStruct((b, L, C), jnp.float32),
    )(tg, mp['lin1']['W'], mp['lin1']['b'])


def _merge_c_body(h_ref, ms_ref, x_ref, c_ref, dwx_ref, dwxc_ref,
                  g_ref, bg_ref, ox_ref, oc_ref, hn_ref):
    cent = c_ref[0]
    m = ms_ref[0, 0, 0]
    sd = ms_ref[0, 0, 1]
    for c in range(_NCH):
        r = slice(c * _CHUNK, (c + 1) * _CHUNK)
        hn = (h_ref[0, r, :] - m) / sd * g_ref[...] + bg_ref[...]
        ox_ref[0, r, :] = hn + x_ref[0, r, :]
        hn_ref[r] = hn
    himg = hn_ref[...].reshape(H, W, C)
    dg0 = jnp.concatenate(
        [himg[:, _DY0[q]:_DY0[q] + 1, :] for q in range(WC)], axis=1)
    dg1 = jnp.concatenate(
        [himg[:, _DY1[q]:_DY1[q] + 1, :] for q in range(WC)], axis=1)
    dwx = dwx_ref[...]
    dwxc = dwxc_ref[...]
    for p in range(HC):
        a = dg0[_DY0[p]]
        b_ = dg1[_DY0[p]]
        c_ = dg0[_DY1[p]]
        d_ = dg1[_DY1[p]]
        top = a * dwxc + b_ * dwx
        bot = c_ * dwxc + d_ * dwx
        rowp = top * _DWYC[p] + bot * _DWY[p]
        oc_ref[0, p * WC:(p + 1) * WC, :] = rowp + cent[p * WC:(p + 1) * WC, :]


def _merge_c(h, msd, xs, cs, mp):
    b = xs.shape[0]
    wxd = jnp.asarray(_DWY.reshape(WC, 1))
    wxcd = jnp.asarray(_DWYC.reshape(WC, 1))
    return pl.pallas_call(
        _merge_c_body,
        grid=(b,),
        compiler_params=_CPARAMS,
        in_specs=[
            pl.BlockSpec((1, L, C), lambda i: (i, 0, 0)),
            pl.BlockSpec((1, 1, 2), lambda i: (i, 0, 0)),
            pl.BlockSpec((1, L, C), lambda i: (i, 0, 0)),
            pl.BlockSpec((1, S, C), lambda i: (i, 0, 0)),
            _full(wxd), _full(wxcd),
            _full(mp['gn']['g']), _full(mp['gn']['b']),
        ],
        out_specs=[
            pl.BlockSpec((1, L, C), lambda i: (i, 0, 0)),
            pl.BlockSpec((1, S, C), lambda i: (i, 0, 0)),
        ],
        out_shape=[
            jax.ShapeDtypeStruct((b, L, C), jnp.float32),
            jax.ShapeDtypeStruct((b, S, C), jnp.float32),
        ],
        scratch_shapes=[pltpu.VMEM((L, C), jnp.float32)],
    )(h, msd, xs, cs, wxd, wxcd, mp['gn']['g'], mp['gn']['b'])


def _cluster_body(x_ref, c_ref, p0_ref, bp0_ref, p1_ref, bp1_ref, ab_ref,
                  wm_ref, bm_ref, g_ref, b_ref, o_ref):
    x = x_ref[0]
    cent = c_ref[0]
    xp = _bdot(x, p0_ref[...]) + bp0_ref[...]
    cp = _bdot(cent, p1_ref[...]) + bp1_ref[...]
    alpha = ab_ref[0]
    beta = ab_ref[1]
    iota = jax.lax.broadcasted_iota(jnp.int32, (L, S), 1)
    cols = []
    for f in range(FC):
        xh = xp[:, f * SC:(f + 1) * SC]
        cph = cp[:, f * 2 * SC:f * 2 * SC + SC]
        cvh = cp[:, f * 2 * SC + SC:(f + 1) * 2 * SC]
        xn = xh / jnp.maximum(
            jnp.sqrt(jnp.sum(xh * xh, axis=1, keepdims=True)), 1e-12)
        cn = cph / jnp.maximum(
            jnp.sqrt(jnp.sum(cph * cph, axis=1, keepdims=True)), 1e-12)
        sim = _bdot(xn, cn, (((1,), (1,)), ((), ())))
        sim = jax.nn.sigmoid(alpha * sim + beta)
        mx = jnp.max(sim, axis=1, keepdims=True)
        idx = jnp.min(jnp.where(sim == mx, iota, S), axis=1, keepdims=True)
        oh = jnp.where(iota == idx, 1.0, 0.0)
        gath = jnp.dot(oh, cvh, precision=_PREC)
        cols.append(mx * gath)
    disp = jnp.concatenate(cols, axis=1)
    o_ref[0] = _bdot(disp, wm_ref[...]) + bm_ref[...]


def _cluster(xs, cc, cl, n0):
    b = xs.shape[0]
    ab = jnp.concatenate([cl['alpha'], cl['beta']])
    return pl.pallas_call(
        _cluster_body,
        grid=(b,),
        compiler_params=_CPARAMS,
        in_specs=[
            pl.BlockSpec((1, L, C), lambda i: (i, 0, 0)),
            pl.BlockSpec((1, S, C), lambda i: (i, 0, 0)),
            _full(cl['proj0']['W']), _full(cl['proj0']['b']),
            _full(cl['proj1']['W']), _full(cl['proj1']['b']),
            _full(ab),
            _full(cl['merge']['W']), _full(cl['merge']['b']),
            _full(n0['g']), _full(n0['b']),
        ],
        out_specs=pl.BlockSpec((1, L, C), lambda i: (i, 0, 0)),
        out_shape=jax.ShapeDtypeStruct((b, L, C), jnp.float32),
    )(xs, cc, cl['proj0']['W'], cl['proj0']['b'],
      cl['proj1']['W'], cl['proj1']['b'], ab,
      cl['merge']['W'], cl['merge']['b'], n0['g'], n0['b'])


def _mlp_a_body(x_ref, n_ref, wl_ref, bl_ref, t_ref):
    for c in range(_NCH):
        r = slice(c * _CHUNK, (c + 1) * _CHUNK)
        t_ref[0, r, :] = _bdot(
            jnp.concatenate([x_ref[0, r, :], n_ref[0, r, :]], axis=1),
            wl_ref[...]) + bl_ref[...]


def _mlp_a(xs, nn, mp):
    b = xs.shape[0]
    wl = mp['lin']['W']
    return pl.pallas_call(
        _mlp_a_body,
        grid=(b,),
        compiler_params=_CPARAMS,
        in_specs=[
            pl.BlockSpec((1, L, C), lambda i: (i, 0, 0)),
            pl.BlockSpec((1, L, C), lambda i: (i, 0, 0)),
            _full(wl), _full(mp['lin']['b']),
        ],
        out_specs=pl.BlockSpec((1, L, 2 * C), lambda i: (i, 0, 0)),
        out_shape=jax.ShapeDtypeStruct((b, L, 2 * C), jnp.float32),
    )(xs, nn, wl, mp['lin']['b'])


def _mlp_b_body(tg_ref, x_ref, wc_ref, bc_ref, g_ref, b_ref, o_ref, tf_ref):
    tf_ref[...] = jnp.zeros((LP + 2, 2 * C), jnp.float32)
    rows = _CHUNK // W
    for c in range(_NCH):
        t = tg_ref[0, c * _CHUNK:(c + 1) * _CHUNK, :]
        for j in range(rows):
            yi = c * rows + j
            base = (yi + 1) * WP + 1
            tf_ref[base:base + W] = t[j * W:(j + 1) * W]
    cw = rows * WP
    for c in range(_NCH):
        a = c * cw
        acc = None
        for k in range(9):
            s = (k // 3) * WP + (k % 3)
            pk = jax.lax.dot_general(
                tf_ref[a + s:a + s + cw], wc_ref[k],
                (((1,), (0,)), ((), ())),
                preferred_element_type=jnp.float32)
            acc = pk if acc is None else acc + pk
        conv = jnp.concatenate(
            [acc[j * WP:j * WP + W] for j in range(rows)], axis=0)
        r = slice(c * _CHUNK, (c + 1) * _CHUNK)
        o_ref[0, r, :] = conv + bc_ref[...]


def _mlp_b(tg, xs, mp, n1):
    b = xs.shape[0]
    wc = mp['conv']['W'].reshape(9, 2 * C, C)
    return pl.pallas_call(
        _mlp_b_body,
        grid=(b,),
        compiler_params=_CPARAMS,
        in_specs=[
            pl.BlockSpec((1, L, 2 * C), lambda i: (i, 0, 0)),
            pl.BlockSpec((1, L, C), lambda i: (i, 0, 0)),
            _full(wc), _full(mp['conv']['b']),
            _full(n1['g']), _full(n1['b']),
        ],
        out_specs=pl.BlockSpec((1, L, C), lambda i: (i, 0, 0)),
        out_shape=jax.ShapeDtypeStruct((b, L, C), jnp.float32),
        scratch_shapes=[pltpu.VMEM((LP + 2, 2 * C), jnp.float32)],
    )(tg, xs, wc, mp['conv']['b'], n1['g'], n1['b'])


def _ln(x, g, b, eps=1e-5):
    m = x.mean(axis=-1, keepdims=True)
    v = x.var(axis=-1, keepdims=True)
    return (x - m) / jnp.sqrt(v + eps) * g + b


def kernel(x0, x1, center0, center1, params, size0, size1):
    xs = jnp.concatenate([x0, x1], axis=0)
    cs = jnp.concatenate([center0, center1], axis=0)
    for i in range(2):
        mp = params['merge'][i]
        gp = params['global'][i]
        t = _merge_a(xs, cs, mp)
        tg = jax.nn.gelu(t, approximate=False)
        h = _merge_b1(tg, mp)
        hh = h.reshape(h.shape[0], H, W, C).transpose(0, 3, 1, 2)
        ms, sds = [], []
        for sl in (slice(0, 2), slice(2, 4)):
            hs = hh[sl]
            ms.append(hs.mean(axis=(1, 2, 3)))
            sds.append(jnp.sqrt(hs.var(axis=(1, 2, 3)) + 1e-5))
        msd = jnp.stack([jnp.concatenate(ms), jnp.concatenate(sds)],
                        axis=1).reshape(-1, 1, 2)
        xs, cs = _merge_c(h, msd, xs, cs, mp)
        cc = cs if i == 0 else jnp.concatenate([cs[2:], cs[:2]], axis=0)
        raw = _cluster(xs, cc, gp['cluster'], gp['norm0'])
        nn = _ln(raw, gp['norm0']['g'], gp['norm0']['b'])
        t2 = _mlp_a(xs, nn, gp['mlp3x3'])
        tg2 = jax.nn.gelu(t2, approximate=False)
        cv = _mlp_b(tg2, xs, gp['mlp3x3'], gp['norm1'])
        xs = _ln(cv, gp['norm1']['g'], gp['norm1']['b']) + xs
    return xs[:2], xs[2:]
